# Initial kernel scaffold; baseline (speedup 1.0000x reference)
import jax
import jax.numpy as jnp
from jax.experimental import pallas as pl


def kernel(vertex_in, face_in, nv_in, mf_in, centers, W0, Wm0, Wv0, Wv1, Wv2, Wv3, Wv4, Wmb0, Wmb1, Wmb2, Wmb3, Wmb4, W_fc1, b_fc1, W_fc2, b_fc2):
    raise NotImplementedError("write your pallas kernel here")


# speedup vs baseline: 1.3654x; 1.3654x over previous
import functools

import jax
import jax.numpy as jnp
from jax import lax
from jax.experimental import pallas as pl
from jax.experimental.pallas import tpu as pltpu
from jax.experimental.pallas import tpu_sc as plsc

N = 50000
F = 100000
B = 8
K = 27
CH = [32, 64, 96, 128, 192, 256]
TAU = 0.1
EPS = 1e-8

NC = 2
NS = 16
NW = NC * NS

FB = 128
N_PAD = 50176
F_PAD = 102400
NBLK = F_PAD // FB
BPT = NBLK // NW
BPS = NBLK // NS
RPT = N_PAD // NS
JUNK = N_PAD - 1

_HIGH = lax.Precision.HIGHEST


def _mesh():
    return plsc.VectorSubcoreMesh(core_axis_name="c", subcore_axis_name="s")


def _sc_gather3(table, fidx):
    D = table.shape[1]

    @functools.partial(
        pl.kernel,
        out_type=jax.ShapeDtypeStruct((3, F_PAD, D), jnp.float32),
        mesh=_mesh(),
        compiler_params=pltpu.CompilerParams(use_tc_tiling_on_sc=False),
        scratch_types=[
            pltpu.VMEM((3, BPT, FB), jnp.int32),
            pltpu.VMEM((FB, D), jnp.float32),
            pltpu.VMEM((FB, D), jnp.float32),
            pltpu.VMEM((FB, D), jnp.float32),
            pltpu.SemaphoreType.DMA,
            pltpu.SemaphoreType.DMA,
            pltpu.SemaphoreType.DMA,
        ],
    )
    def k(table_h, fidx_h, out_h, idx_v, b0, b1, b2, s0, s1, s2):
        cid = lax.axis_index("c")
        sid = lax.axis_index("s")
        wid = cid * NS + sid
        pltpu.sync_copy(fidx_h.at[:, pl.ds(wid * BPT, BPT), :], idx_v)
        fbase = wid * BPT * FB

        def body(b, carry):
            cp0 = pltpu.async_copy(table_h.at[idx_v.at[0, b]], b0, s0)
            cp1 = pltpu.async_copy(table_h.at[idx_v.at[1, b]], b1, s1)
            cp2 = pltpu.async_copy(table_h.at[idx_v.at[2, b]], b2, s2)
            cp0.wait()
            cp1.wait()
            cp2.wait()
            row0 = fbase + b * FB
            pltpu.sync_copy(b0, out_h.at[0, pl.ds(row0, FB), :])
            pltpu.sync_copy(b1, out_h.at[1, pl.ds(row0, FB), :])
            pltpu.sync_copy(b2, out_h.at[2, pl.ds(row0, FB), :])
            return carry

        lax.fori_loop(0, BPT, body, 0)

    return k(table, fidx)


def _sc_count(fidx):
    CW = 16

    @functools.partial(
        pl.kernel,
        out_type=jax.ShapeDtypeStruct((NC, N_PAD, CW), jnp.float32),
        mesh=_mesh(),
        compiler_params=pltpu.CompilerParams(use_tc_tiling_on_sc=False),
        scratch_types=[
            pltpu.VMEM((3, BPT, FB), jnp.int32),
            pltpu.VMEM((FB, CW), jnp.float32),
            pltpu.VMEM((784, CW), jnp.float32),
            pltpu.VMEM_SHARED((N_PAD, CW), jnp.float32),
        ],
    )
    def k(fidx_h, cnt_h, idx_v, ones_v, z_v, acc):
        cid = lax.axis_index("c")
        sid = lax.axis_index("s")
        wid = cid * NS + sid
        pltpu.sync_copy(fidx_h.at[:, pl.ds(wid * BPT, BPT), :], idx_v)

        one = jnp.ones((16,), jnp.float32)
        zero = jnp.zeros((16,), jnp.float32)

        def fill_ones(i, carry):
            ones_v[i, pl.ds(0, 16)] = one
            return carry

        lax.fori_loop(0, FB, fill_ones, 0)

        def fill_zero(i, carry):
            z_v[i, pl.ds(0, 16)] = zero
            return carry

        lax.fori_loop(0, 784, fill_zero, 0)

        for r in range(RPT // 784):
            pltpu.sync_copy(z_v, acc.at[pl.ds(sid * RPT + r * 784, 784), :])
        plsc.subcore_barrier()

        def body(b, carry):
            pltpu.sync_copy(ones_v, acc.at[idx_v.at[0, b]], add=True)
            pltpu.sync_copy(ones_v, acc.at[idx_v.at[1, b]], add=True)
            pltpu.sync_copy(ones_v, acc.at[idx_v.at[2, b]], add=True)
            return carry

        lax.fori_loop(0, BPT, body, 0)
        plsc.subcore_barrier()

        pltpu.sync_copy(acc.at[pl.ds(sid * RPT, RPT), :],
                        cnt_h.at[cid, pl.ds(sid * RPT, RPT), :])

    return k(fidx)


def _sc_scatter3(t, fidx, cout):
    WC = 16
    n_chunks = cout // WC
    npass = (n_chunks + 1) // 2

    @functools.partial(
        pl.kernel,
        out_type=jax.ShapeDtypeStruct((N_PAD, cout), jnp.float32),
        mesh=_mesh(),
        compiler_params=pltpu.CompilerParams(use_tc_tiling_on_sc=False),
        scratch_types=[
            pltpu.VMEM((3, BPS, FB), jnp.int32),
            pltpu.VMEM((FB, WC), jnp.float32),
            pltpu.VMEM((784, WC), jnp.float32),
            pltpu.VMEM_SHARED((N_PAD, WC), jnp.float32),
        ],
    )
    def k(t_h, fidx_h, out_h, idx_v, tbuf, z_v, acc):
        cid = lax.axis_index("c")
        sid = lax.axis_index("s")
        pltpu.sync_copy(fidx_h.at[:, pl.ds(sid * BPS, BPS), :], idx_v)
        fbase = sid * BPS * FB

        zero = jnp.zeros((16,), jnp.float32)

        def fill_zero(i, carry):
            for q in range(WC // 16):
                z_v[i, pl.ds(q * 16, 16)] = zero
            return carry

        lax.fori_loop(0, 784, fill_zero, 0)

        for jj in range(npass):
            j = jj * 2 + cid

            @pl.when(j < n_chunks)
            def _pass():
                c0 = j * WC
                for r in range(RPT // 784):
                    pltpu.sync_copy(z_v, acc.at[pl.ds(sid * RPT + r * 784, 784), :])
                plsc.subcore_barrier()

                def body(b, carry):
                    pltpu.sync_copy(
                        t_h.at[pl.ds(fbase + b * FB, FB), pl.ds(c0, WC)], tbuf)
                    pltpu.sync_copy(tbuf, acc.at[idx_v.at[0, b]], add=True)
                    pltpu.sync_copy(tbuf, acc.at[idx_v.at[1, b]], add=True)
                    pltpu.sync_copy(tbuf, acc.at[idx_v.at[2, b]], add=True)
                    return carry

                lax.fori_loop(0, BPS, body, 0)
                plsc.subcore_barrier()
                pltpu.sync_copy(
                    acc.at[pl.ds(sid * RPT, RPT), :],
                    out_h.at[pl.ds(sid * RPT, RPT), pl.ds(c0, WC)])
                plsc.subcore_barrier()

    return k(t, fidx)


def _tc_inv(cnt):
    def body(c_ref, o_ref):
        c = c_ref[0] + c_ref[1]
        o_ref[...] = 1.0 / jnp.maximum(c, 1.0)

    return pl.pallas_call(
        body,
        out_shape=jax.ShapeDtypeStruct((392, 128), jnp.float32),
    )(cnt)


def _tc_geom(va, vb, vc, centers_p, w0_p, wm0_p):
    BF = 512

    def body(va_ref, vb_ref, vc_ref, cen_ref, w0_ref, wm_ref, t0_ref, cf_ref):
        V1 = va_ref[:, 0:3]
        V2 = vb_ref[:, 0:3]
        V3 = vc_ref[:, 0:3]
        D12 = V2 - V1
        D23 = V3 - V2
        D31 = V1 - V3
        L12 = jnp.sqrt(jnp.sum(D12 * D12, axis=-1, keepdims=True))
        L23 = jnp.sqrt(jnp.sum(D23 * D23, axis=-1, keepdims=True))
        L31 = jnp.sqrt(jnp.sum(D31 * D31, axis=-1, keepdims=True))
        Bv = -D31
        ax, ay, az = D12[:, 0:1], D12[:, 1:2], D12[:, 2:3]
        bx, by, bz = Bv[:, 0:1], Bv[:, 1:2], Bv[:, 2:3]
        nrm = jnp.concatenate(
            [ay * bz - az * by, az * bx - ax * bz, ax * by - ay * bx], axis=-1)
        nl = jnp.sqrt(jnp.sum(nrm * nrm, axis=-1, keepdims=True))
        normals = nrm / (nl + EPS)
        T1 = jnp.sum(D12 * -D31, axis=-1, keepdims=True) / (L12 * L31 + EPS)
        T2 = jnp.sum(-D12 * D23, axis=-1, keepdims=True) / (L12 * L23 + EPS)
        T3 = jnp.sum(-D23 * D31, axis=-1, keepdims=True) / (L23 * L31 + EPS)
        zeros7 = jnp.zeros((BF, 7), jnp.float32)
        geo16 = jnp.concatenate(
            [L12, L23, L31, T1, T2, T3, normals, zeros7], axis=-1)
        n8 = jnp.concatenate([normals, jnp.zeros((BF, 5), jnp.float32)], axis=-1)
        logits = jnp.dot(n8, cen_ref[...], precision=_HIGH) * (1.0 / TAU)
        col = lax.broadcasted_iota(jnp.int32, (BF, 32), 1)
        logits = jnp.where(col < K, logits, -1e30)
        m = jnp.max(logits, axis=-1, keepdims=True)
        e = jnp.exp(logits - m)
        coeff = e / jnp.sum(e, axis=-1, keepdims=True)
        hf0 = jnp.maximum(jnp.dot(geo16, w0_ref[...], precision=_HIGH), 0.0)
        scale = jnp.dot(coeff, wm_ref[...], precision=_HIGH)
        t0_ref[...] = hf0 * scale
        cf_ref[...] = coeff

    grid = (F_PAD // BF,)
    return pl.pallas_call(
        body,
        grid=grid,
        in_specs=[
            pl.BlockSpec((BF, 8), lambda i: (i, 0)),
            pl.BlockSpec((BF, 8), lambda i: (i, 0)),
            pl.BlockSpec((BF, 8), lambda i: (i, 0)),
            pl.BlockSpec((8, 32), lambda i: (0, 0)),
            pl.BlockSpec((16, 32), lambda i: (0, 0)),
            pl.BlockSpec((32, 32), lambda i: (0, 0)),
        ],
        out_specs=[
            pl.BlockSpec((BF, 32), lambda i: (i, 0)),
            pl.BlockSpec((BF, 32), lambda i: (i, 0)),
        ],
        out_shape=[
            jax.ShapeDtypeStruct((F_PAD, 32), jnp.float32),
            jax.ShapeDtypeStruct((F_PAD, 32), jnp.float32),
        ],
    )(va, vb, vc, centers_p, w0_p, wm0_p)


def _tc_layer(ga, gb, gc, coeff, wv_s, wm_p):
    BF = 256
    cin = ga.shape[1]
    cout = wv_s.shape[1]

    def body(ga_ref, gb_ref, gc_ref, cf_ref, wv_ref, wm_ref, t_ref):
        g = ga_ref[...] + gb_ref[...] + gc_ref[...]
        hf = jnp.maximum(jnp.dot(g, wv_ref[...], precision=_HIGH), 0.0)
        scale = jnp.dot(cf_ref[...], wm_ref[...], precision=_HIGH)
        t_ref[...] = hf * scale

    return pl.pallas_call(
        body,
        grid=(F_PAD // BF,),
        in_specs=[
            pl.BlockSpec((BF, cin), lambda i: (i, 0)),
            pl.BlockSpec((BF, cin), lambda i: (i, 0)),
            pl.BlockSpec((BF, cin), lambda i: (i, 0)),
            pl.BlockSpec((BF, 32), lambda i: (i, 0)),
            pl.BlockSpec((cin, cout), lambda i: (0, 0)),
            pl.BlockSpec((32, cout), lambda i: (0, 0)),
        ],
        out_specs=pl.BlockSpec((BF, cout), lambda i: (i, 0)),
        out_shape=jax.ShapeDtypeStruct((F_PAD, cout), jnp.float32),
    )(ga, gb, gc, coeff, wv_s, wm_p)


def _tc_feats(acc, inv):
    BV = 512
    c = acc.shape[1]

    def body(a_ref, i_ref, o_ref):
        o_ref[...] = jnp.maximum(a_ref[...], 0.0) * i_ref[...]

    return pl.pallas_call(
        body,
        grid=(N_PAD // BV,),
        in_specs=[
            pl.BlockSpec((BV, c), lambda i: (i, 0)),
            pl.BlockSpec((BV, 1), lambda i: (i, 0)),
        ],
        out_specs=pl.BlockSpec((BV, c), lambda i: (i, 0)),
        out_shape=jax.ShapeDtypeStruct((N_PAD, c), jnp.float32),
    )(acc, inv)


def _tc_pool(feats3):
    seg = N // B

    def body(f_ref, o_ref):
        o_ref[0, 0, :] = jnp.sum(f_ref[0], axis=0) * (1.0 / seg)

    return pl.pallas_call(
        body,
        grid=(B,),
        in_specs=[pl.BlockSpec((1, seg, 256), lambda i: (i, 0, 0))],
        out_specs=pl.BlockSpec((1, 1, 256), lambda i: (i, 0, 0)),
        out_shape=jax.ShapeDtypeStruct((B, 1, 256), jnp.float32),
    )(feats3)


def _tc_fc(pooled, w1, b1, w2, b2):
    def body(p_ref, w1_ref, b1_ref, w2_ref, b2_ref, o_ref):
        h = jnp.maximum(
            jnp.dot(p_ref[...], w1_ref[...], precision=_HIGH) + b1_ref[...], 0.0)
        o_ref[...] = jnp.dot(h, w2_ref[...], precision=_HIGH) + b2_ref[...]

    return pl.pallas_call(
        body,
        out_shape=jax.ShapeDtypeStruct((B, 20), jnp.float32),
    )(pooled, w1, b1, w2, b2)


def kernel(vertex_in, face_in, nv_in, mf_in, centers, W0, Wm0, Wv0, Wv1, Wv2,
           Wv3, Wv4, Wmb0, Wmb1, Wmb2, Wmb3, Wmb4, W_fc1, b_fc1, W_fc2, b_fc2):
    face32 = face_in.astype(jnp.int32)
    pad_idx = jnp.full((F_PAD - F, 3), JUNK, jnp.int32)
    fidx = jnp.concatenate([face32, pad_idx], axis=0).T.reshape(3, NBLK, FB)
    vert_p = jnp.pad(vertex_in, ((0, N_PAD - N), (0, 5)))

    centers_p = jnp.zeros((8, 32), jnp.float32).at[0:3, 0:K].set(centers.T)
    w0_p = jnp.pad(W0, ((0, 7), (0, 0)))
    wm0_p = jnp.pad(Wm0, ((0, 32 - K), (0, 0)))
    wvs = [w / 3.0 for w in (Wv0, Wv1, Wv2, Wv3, Wv4)]
    wms = [jnp.pad(w, ((0, 32 - K), (0, 0))) for w in (Wmb0, Wmb1, Wmb2, Wmb3, Wmb4)]

    cnt = _sc_count(fidx)[:, :, 0]
    inv = _tc_inv(cnt.reshape(2, 392, 128)).reshape(N_PAD, 1)

    g3v = _sc_gather3(vert_p, fidx)
    t0, coeff = _tc_geom(g3v[0], g3v[1], g3v[2], centers_p, w0_p, wm0_p)

    acc = _sc_scatter3(t0, fidx, CH[0])
    feats = _tc_feats(acc, inv)
    for l in range(5):
        g3 = _sc_gather3(feats, fidx)
        t = _tc_layer(g3[0], g3[1], g3[2], coeff, wvs[l], wms[l])
        acc = _sc_scatter3(t, fidx, CH[l + 1])
        feats = _tc_feats(acc, inv)

    pooled = _tc_pool(feats[:N].reshape(B, N // B, 256)).reshape(B, 256)
    logits = _tc_fc(pooled, W_fc1, b_fc1.reshape(1, 128), W_fc2,
                    b_fc2.reshape(1, 20))
    return logits


# speedup vs baseline: 1.4617x; 1.0706x over previous
import functools

import jax
import jax.numpy as jnp
from jax import lax
from jax.experimental import pallas as pl
from jax.experimental.pallas import tpu as pltpu
from jax.experimental.pallas import tpu_sc as plsc

N = 50000
F = 100000
B = 8
K = 27
CH = [32, 64, 96, 128, 192, 256]
TAU = 0.1
EPS = 1e-8

NC = 2
NS = 16
NW = NC * NS

FB = 128
N_PAD = 50176
F_PAD = 102400
NBLK = F_PAD // FB
BPT = NBLK // NW
BPS = NBLK // NS
RPT = N_PAD // NS
JUNK = N_PAD - 1

_HIGH = lax.Precision.HIGHEST


def _mesh():
    return plsc.VectorSubcoreMesh(core_axis_name="c", subcore_axis_name="s")


def _sc_gather3(table, fidx):
    D = table.shape[1]

    SPB = 1 if D <= 128 else 2
    FBG = FB // SPB
    U = BPT * SPB
    KB = (U + 1) // 2

    @functools.partial(
        pl.kernel,
        out_type=jax.ShapeDtypeStruct((3, F_PAD, D), jnp.float32),
        mesh=_mesh(),
        compiler_params=pltpu.CompilerParams(use_tc_tiling_on_sc=False),
        scratch_types=[
            pltpu.VMEM((3, BPT, FB), jnp.int32),
            pltpu.VMEM((2, 3, FBG, D), jnp.float32),
            pltpu.SemaphoreType.DMA,
            pltpu.SemaphoreType.DMA,
            pltpu.SemaphoreType.DMA,
            pltpu.SemaphoreType.DMA,
        ],
    )
    def k(table_h, fidx_h, out_h, idx_v, buf, g0, g1, w0, w1):
        cid = lax.axis_index("c")
        sid = lax.axis_index("s")
        wid = cid * NS + sid
        pltpu.sync_copy(fidx_h.at[:, pl.ds(wid * BPT, BPT), :], idx_v)
        fbase = wid * BPT * FB
        gsem = (g0, g1)
        wsem = (w0, w1)

        def idx_at(c, b, h):
            if SPB == 1:
                return idx_v.at[c, b]
            return idx_v.at[c, b, pl.ds(h * FBG, FBG)]

        def gather(u, b, h, p):
            for c in range(3):
                pltpu.async_copy(table_h.at[idx_at(c, b, h)], buf.at[p, c],
                                 gsem[p])

        def writeback(u, p):
            row0 = fbase + u * FBG
            for c in range(3):
                pltpu.async_copy(buf.at[p, c], out_h.at[c, pl.ds(row0, FBG), :],
                                 wsem[p])

        def wait_g(p):
            for c in range(3):
                pltpu.make_async_copy(table_h.at[pl.ds(0, FBG)], buf.at[p, c],
                                      gsem[p]).wait()

        def wait_w(p):
            for c in range(3):
                pltpu.make_async_copy(buf.at[p, c],
                                      out_h.at[c, pl.ds(0, FBG), :],
                                      wsem[p]).wait()

        gather(0, 0, 0, 0)

        def body(kk, carry):
            u = 2 * kk
            wait_g(0)

            @pl.when(kk > 0)
            def _():
                wait_w(1)

            @pl.when(u + 1 < U)
            def _():
                b1, h1 = (u + 1, 0) if SPB == 1 else (kk, 1)
                gather(u + 1, b1, h1, 1)

            writeback(u, 0)

            @pl.when(u + 1 < U)
            def _():
                wait_g(1)
                wait_w(0)

                @pl.when(u + 2 < U)
                def _():
                    b2, h2 = (u + 2, 0) if SPB == 1 else (kk + 1, 0)
                    gather(u + 2, b2, h2, 0)

                writeback(u + 1, 1)

            return carry

        lax.fori_loop(0, KB, body, 0)
        if U % 2 == 1:
            wait_w(0)
        else:
            wait_w(1)

    return k(table, fidx)


def _sc_count(fidx):
    CW = 16

    @functools.partial(
        pl.kernel,
        out_type=jax.ShapeDtypeStruct((NC, N_PAD, CW), jnp.float32),
        mesh=_mesh(),
        compiler_params=pltpu.CompilerParams(use_tc_tiling_on_sc=False),
        scratch_types=[
            pltpu.VMEM((3, BPT, FB), jnp.int32),
            pltpu.VMEM((FB, CW), jnp.float32),
            pltpu.VMEM((784, CW), jnp.float32),
            pltpu.VMEM_SHARED((N_PAD, CW), jnp.float32),
        ],
    )
    def k(fidx_h, cnt_h, idx_v, ones_v, z_v, acc):
        cid = lax.axis_index("c")
        sid = lax.axis_index("s")
        wid = cid * NS + sid
        pltpu.sync_copy(fidx_h.at[:, pl.ds(wid * BPT, BPT), :], idx_v)

        one = jnp.ones((16,), jnp.float32)
        zero = jnp.zeros((16,), jnp.float32)

        def fill_ones(i, carry):
            ones_v[i, pl.ds(0, 16)] = one
            return carry

        lax.fori_loop(0, FB, fill_ones, 0)

        def fill_zero(i, carry):
            z_v[i, pl.ds(0, 16)] = zero
            return carry

        lax.fori_loop(0, 784, fill_zero, 0)

        for r in range(RPT // 784):
            pltpu.sync_copy(z_v, acc.at[pl.ds(sid * RPT + r * 784, 784), :])
        plsc.subcore_barrier()

        def body(b, carry):
            pltpu.sync_copy(ones_v, acc.at[idx_v.at[0, b]], add=True)
            pltpu.sync_copy(ones_v, acc.at[idx_v.at[1, b]], add=True)
            pltpu.sync_copy(ones_v, acc.at[idx_v.at[2, b]], add=True)
            return carry

        lax.fori_loop(0, BPT, body, 0)
        plsc.subcore_barrier()

        pltpu.sync_copy(acc.at[pl.ds(sid * RPT, RPT), :],
                        cnt_h.at[cid, pl.ds(sid * RPT, RPT), :])

    return k(fidx)


def _sc_scatter3(t, fidx, cout):
    WC = 16
    n_chunks = cout // WC
    npass = (n_chunks + 1) // 2

    @functools.partial(
        pl.kernel,
        out_type=jax.ShapeDtypeStruct((N_PAD, cout), jnp.float32),
        mesh=_mesh(),
        compiler_params=pltpu.CompilerParams(use_tc_tiling_on_sc=False),
        scratch_types=[
            pltpu.VMEM((3, BPS, FB), jnp.int32),
            pltpu.VMEM((2, FB, WC), jnp.float32),
            pltpu.VMEM((784, WC), jnp.float32),
            pltpu.VMEM_SHARED((N_PAD, WC), jnp.float32),
            pltpu.SemaphoreType.DMA,
            pltpu.SemaphoreType.DMA,
            pltpu.SemaphoreType.DMA,
            pltpu.SemaphoreType.DMA,
        ],
    )
    def k(t_h, fidx_h, out_h, idx_v, tbuf, z_v, acc, l0, l1, a0, a1):
        cid = lax.axis_index("c")
        sid = lax.axis_index("s")
        pltpu.sync_copy(fidx_h.at[:, pl.ds(sid * BPS, BPS), :], idx_v)
        fbase = sid * BPS * FB
        lsem = (l0, l1)
        asem = (a0, a1)

        zero = jnp.zeros((16,), jnp.float32)

        def fill_zero(i, carry):
            for q in range(WC // 16):
                z_v[i, pl.ds(q * 16, 16)] = zero
            return carry

        lax.fori_loop(0, 784, fill_zero, 0)

        for jj in range(npass):
            j = jj * 2 + cid

            @pl.when(j < n_chunks)
            def _pass():
                c0 = j * WC
                for r in range(RPT // 784):
                    pltpu.sync_copy(z_v, acc.at[pl.ds(sid * RPT + r * 784, 784), :])
                plsc.subcore_barrier()

                def load(b, p):
                    pltpu.async_copy(
                        t_h.at[pl.ds(fbase + b * FB, FB), pl.ds(c0, WC)],
                        tbuf.at[p], lsem[p])

                def adds(b, p):
                    for c in range(3):
                        pltpu.async_copy(tbuf.at[p], acc.at[idx_v.at[c, b]],
                                         asem[p], add=True)

                def wait_l(p):
                    pltpu.make_async_copy(
                        t_h.at[pl.ds(0, FB), pl.ds(0, WC)], tbuf.at[p],
                        lsem[p]).wait()

                def wait_a(p):
                    for c in range(3):
                        pltpu.make_async_copy(tbuf.at[p],
                                              acc.at[pl.ds(0, FB), :],
                                              asem[p]).wait()

                load(0, 0)

                def body(kk, carry):
                    u = 2 * kk
                    wait_l(0)

                    @pl.when(kk > 0)
                    def _():
                        wait_a(1)

                    load(u + 1, 1)
                    adds(u, 0)
                    wait_l(1)
                    wait_a(0)

                    @pl.when(u + 2 < BPS)
                    def _():
                        load(u + 2, 0)

                    adds(u + 1, 1)
                    return carry

                lax.fori_loop(0, BPS // 2, body, 0)
                wait_a(1)
                plsc.subcore_barrier()
                pltpu.sync_copy(
                    acc.at[pl.ds(sid * RPT, RPT), :],
                    out_h.at[pl.ds(sid * RPT, RPT), pl.ds(c0, WC)])
                plsc.subcore_barrier()

    return k(t, fidx)


def _tc_inv(cnt):
    def body(c_ref, o_ref):
        c = c_ref[0] + c_ref[1]
        o_ref[...] = 1.0 / jnp.maximum(c, 1.0)

    return pl.pallas_call(
        body,
        out_shape=jax.ShapeDtypeStruct((392, 128), jnp.float32),
    )(cnt)


def _tc_geom(va, vb, vc, centers_p, w0_p, wm0_p):
    BF = 512

    def body(va_ref, vb_ref, vc_ref, cen_ref, w0_ref, wm_ref, t0_ref, cf_ref):
        V1 = va_ref[:, 0:3]
        V2 = vb_ref[:, 0:3]
        V3 = vc_ref[:, 0:3]
        D12 = V2 - V1
        D23 = V3 - V2
        D31 = V1 - V3
        L12 = jnp.sqrt(jnp.sum(D12 * D12, axis=-1, keepdims=True))
        L23 = jnp.sqrt(jnp.sum(D23 * D23, axis=-1, keepdims=True))
        L31 = jnp.sqrt(jnp.sum(D31 * D31, axis=-1, keepdims=True))
        Bv = -D31
        ax, ay, az = D12[:, 0:1], D12[:, 1:2], D12[:, 2:3]
        bx, by, bz = Bv[:, 0:1], Bv[:, 1:2], Bv[:, 2:3]
        nrm = jnp.concatenate(
            [ay * bz - az * by, az * bx - ax * bz, ax * by - ay * bx], axis=-1)
        nl = jnp.sqrt(jnp.sum(nrm * nrm, axis=-1, keepdims=True))
        normals = nrm / (nl + EPS)
        T1 = jnp.sum(D12 * -D31, axis=-1, keepdims=True) / (L12 * L31 + EPS)
        T2 = jnp.sum(-D12 * D23, axis=-1, keepdims=True) / (L12 * L23 + EPS)
        T3 = jnp.sum(-D23 * D31, axis=-1, keepdims=True) / (L23 * L31 + EPS)
        zeros7 = jnp.zeros((BF, 7), jnp.float32)
        geo16 = jnp.concatenate(
            [L12, L23, L31, T1, T2, T3, normals, zeros7], axis=-1)
        n8 = jnp.concatenate([normals, jnp.zeros((BF, 5), jnp.float32)], axis=-1)
        logits = jnp.dot(n8, cen_ref[...], precision=_HIGH) * (1.0 / TAU)
        col = lax.broadcasted_iota(jnp.int32, (BF, 32), 1)
        logits = jnp.where(col < K, logits, -1e30)
        m = jnp.max(logits, axis=-1, keepdims=True)
        e = jnp.exp(logits - m)
        coeff = e / jnp.sum(e, axis=-1, keepdims=True)
        hf0 = jnp.maximum(jnp.dot(geo16, w0_ref[...], precision=_HIGH), 0.0)
        scale = jnp.dot(coeff, wm_ref[...], precision=_HIGH)
        t0_ref[...] = hf0 * scale
        cf_ref[...] = coeff

    grid = (F_PAD // BF,)
    return pl.pallas_call(
        body,
        grid=grid,
        in_specs=[
            pl.BlockSpec((BF, 8), lambda i: (i, 0)),
            pl.BlockSpec((BF, 8), lambda i: (i, 0)),
            pl.BlockSpec((BF, 8), lambda i: (i, 0)),
            pl.BlockSpec((8, 32), lambda i: (0, 0)),
            pl.BlockSpec((16, 32), lambda i: (0, 0)),
            pl.BlockSpec((32, 32), lambda i: (0, 0)),
        ],
        out_specs=[
            pl.BlockSpec((BF, 32), lambda i: (i, 0)),
            pl.BlockSpec((BF, 32), lambda i: (i, 0)),
        ],
        out_shape=[
            jax.ShapeDtypeStruct((F_PAD, 32), jnp.float32),
            jax.ShapeDtypeStruct((F_PAD, 32), jnp.float32),
        ],
    )(va, vb, vc, centers_p, w0_p, wm0_p)


def _tc_layer(ga, gb, gc, coeff, wv_s, wm_p):
    BF = 256
    cin = ga.shape[1]
    cout = wv_s.shape[1]

    def body(ga_ref, gb_ref, gc_ref, cf_ref, wv_ref, wm_ref, t_ref):
        g = ga_ref[...] + gb_ref[...] + gc_ref[...]
        hf = jnp.maximum(jnp.dot(g, wv_ref[...], precision=_HIGH), 0.0)
        scale = jnp.dot(cf_ref[...], wm_ref[...], precision=_HIGH)
        t_ref[...] = hf * scale

    return pl.pallas_call(
        body,
        grid=(F_PAD // BF,),
        in_specs=[
            pl.BlockSpec((BF, cin), lambda i: (i, 0)),
            pl.BlockSpec((BF, cin), lambda i: (i, 0)),
            pl.BlockSpec((BF, cin), lambda i: (i, 0)),
            pl.BlockSpec((BF, 32), lambda i: (i, 0)),
            pl.BlockSpec((cin, cout), lambda i: (0, 0)),
            pl.BlockSpec((32, cout), lambda i: (0, 0)),
        ],
        out_specs=pl.BlockSpec((BF, cout), lambda i: (i, 0)),
        out_shape=jax.ShapeDtypeStruct((F_PAD, cout), jnp.float32),
    )(ga, gb, gc, coeff, wv_s, wm_p)


def _tc_feats(acc, inv):
    BV = 512
    c = acc.shape[1]

    def body(a_ref, i_ref, o_ref):
        o_ref[...] = jnp.maximum(a_ref[...], 0.0) * i_ref[...]

    return pl.pallas_call(
        body,
        grid=(N_PAD // BV,),
        in_specs=[
            pl.BlockSpec((BV, c), lambda i: (i, 0)),
            pl.BlockSpec((BV, 1), lambda i: (i, 0)),
        ],
        out_specs=pl.BlockSpec((BV, c), lambda i: (i, 0)),
        out_shape=jax.ShapeDtypeStruct((N_PAD, c), jnp.float32),
    )(acc, inv)


def _tc_pool(feats3):
    seg = N // B

    def body(f_ref, o_ref):
        o_ref[0, 0, :] = jnp.sum(f_ref[0], axis=0) * (1.0 / seg)

    return pl.pallas_call(
        body,
        grid=(B,),
        in_specs=[pl.BlockSpec((1, seg, 256), lambda i: (i, 0, 0))],
        out_specs=pl.BlockSpec((1, 1, 256), lambda i: (i, 0, 0)),
        out_shape=jax.ShapeDtypeStruct((B, 1, 256), jnp.float32),
    )(feats3)


def _tc_fc(pooled, w1, b1, w2, b2):
    def body(p_ref, w1_ref, b1_ref, w2_ref, b2_ref, o_ref):
        h = jnp.maximum(
            jnp.dot(p_ref[...], w1_ref[...], precision=_HIGH) + b1_ref[...], 0.0)
        o_ref[...] = jnp.dot(h, w2_ref[...], precision=_HIGH) + b2_ref[...]

    return pl.pallas_call(
        body,
        out_shape=jax.ShapeDtypeStruct((B, 20), jnp.float32),
    )(pooled, w1, b1, w2, b2)


def kernel(vertex_in, face_in, nv_in, mf_in, centers, W0, Wm0, Wv0, Wv1, Wv2,
           Wv3, Wv4, Wmb0, Wmb1, Wmb2, Wmb3, Wmb4, W_fc1, b_fc1, W_fc2, b_fc2):
    face32 = face_in.astype(jnp.int32)
    pad_idx = jnp.full((F_PAD - F, 3), JUNK, jnp.int32)
    fidx = jnp.concatenate([face32, pad_idx], axis=0).T.reshape(3, NBLK, FB)
    vert_p = jnp.pad(vertex_in, ((0, N_PAD - N), (0, 5)))

    centers_p = jnp.zeros((8, 32), jnp.float32).at[0:3, 0:K].set(centers.T)
    w0_p = jnp.pad(W0, ((0, 7), (0, 0)))
    wm0_p = jnp.pad(Wm0, ((0, 32 - K), (0, 0)))
    wvs = [w / 3.0 for w in (Wv0, Wv1, Wv2, Wv3, Wv4)]
    wms = [jnp.pad(w, ((0, 32 - K), (0, 0))) for w in (Wmb0, Wmb1, Wmb2, Wmb3, Wmb4)]

    cnt = _sc_count(fidx)[:, :, 0]
    inv = _tc_inv(cnt.reshape(2, 392, 128)).reshape(N_PAD, 1)

    g3v = _sc_gather3(vert_p, fidx)
    t0, coeff = _tc_geom(g3v[0], g3v[1], g3v[2], centers_p, w0_p, wm0_p)

    acc = _sc_scatter3(t0, fidx, CH[0])
    feats = _tc_feats(acc, inv)
    for l in range(5):
        g3 = _sc_gather3(feats, fidx)
        t = _tc_layer(g3[0], g3[1], g3[2], coeff, wvs[l], wms[l])
        acc = _sc_scatter3(t, fidx, CH[l + 1])
        feats = _tc_feats(acc, inv)

    pooled = _tc_pool(feats[:N].reshape(B, N // B, 256)).reshape(B, 256)
    logits = _tc_fc(pooled, W_fc1, b_fc1.reshape(1, 128), W_fc2,
                    b_fc2.reshape(1, 20))
    return logits


# speedup vs baseline: 1.5746x; 1.0772x over previous
import functools

import jax
import jax.numpy as jnp
from jax import lax
from jax.experimental import pallas as pl
from jax.experimental.pallas import tpu as pltpu
from jax.experimental.pallas import tpu_sc as plsc

N = 50000
F = 100000
B = 8
K = 27
CH = [32, 64, 96, 128, 192, 256]
TAU = 0.1
EPS = 1e-8

NC = 2
NS = 16
NW = NC * NS

FB = 128
N_PAD = 50176
F_PAD = 102400
NBLK = F_PAD // FB
BPT = NBLK // NW
BPS = NBLK // NS
RPT = N_PAD // NS
JUNK = N_PAD - 1

_HIGH = lax.Precision.HIGHEST


def _mesh():
    return plsc.VectorSubcoreMesh(core_axis_name="c", subcore_axis_name="s")


def _sc_gather3(table, fidx):
    D = table.shape[1]

    SPB = 1 if D <= 128 else 2
    FBG = FB // SPB
    U = BPT * SPB
    KB = (U + 1) // 2

    @functools.partial(
        pl.kernel,
        out_type=jax.ShapeDtypeStruct((3, F_PAD, D), jnp.float32),
        mesh=_mesh(),
        compiler_params=pltpu.CompilerParams(use_tc_tiling_on_sc=False),
        scratch_types=[
            pltpu.VMEM((3, BPT, FB), jnp.int32),
            pltpu.VMEM((2, 3, FBG, D), jnp.float32),
            pltpu.SemaphoreType.DMA,
            pltpu.SemaphoreType.DMA,
            pltpu.SemaphoreType.DMA,
            pltpu.SemaphoreType.DMA,
        ],
    )
    def k(table_h, fidx_h, out_h, idx_v, buf, g0, g1, w0, w1):
        cid = lax.axis_index("c")
        sid = lax.axis_index("s")
        wid = cid * NS + sid
        pltpu.sync_copy(fidx_h.at[:, pl.ds(wid * BPT, BPT), :], idx_v)
        fbase = wid * BPT * FB
        gsem = (g0, g1)
        wsem = (w0, w1)

        def idx_at(c, b, h):
            if SPB == 1:
                return idx_v.at[c, b]
            return idx_v.at[c, b, pl.ds(h * FBG, FBG)]

        def gather(u, b, h, p):
            for c in range(3):
                pltpu.async_copy(table_h.at[idx_at(c, b, h)], buf.at[p, c],
                                 gsem[p])

        def writeback(u, p):
            row0 = fbase + u * FBG
            for c in range(3):
                pltpu.async_copy(buf.at[p, c], out_h.at[c, pl.ds(row0, FBG), :],
                                 wsem[p])

        def wait_g(p):
            for c in range(3):
                pltpu.make_async_copy(table_h.at[pl.ds(0, FBG)], buf.at[p, c],
                                      gsem[p]).wait()

        def wait_w(p):
            for c in range(3):
                pltpu.make_async_copy(buf.at[p, c],
                                      out_h.at[c, pl.ds(0, FBG), :],
                                      wsem[p]).wait()

        gather(0, 0, 0, 0)

        def body(kk, carry):
            u = 2 * kk
            wait_g(0)

            @pl.when(kk > 0)
            def _():
                wait_w(1)

            @pl.when(u + 1 < U)
            def _():
                b1, h1 = (u + 1, 0) if SPB == 1 else (kk, 1)
                gather(u + 1, b1, h1, 1)

            writeback(u, 0)

            @pl.when(u + 1 < U)
            def _():
                wait_g(1)
                wait_w(0)

                @pl.when(u + 2 < U)
                def _():
                    b2, h2 = (u + 2, 0) if SPB == 1 else (kk + 1, 0)
                    gather(u + 2, b2, h2, 0)

                writeback(u + 1, 1)

            return carry

        lax.fori_loop(0, KB, body, 0)
        if U % 2 == 1:
            wait_w(0)
        else:
            wait_w(1)

    return k(table, fidx)


def _sc_count(fidx):
    CW = 16

    @functools.partial(
        pl.kernel,
        out_type=jax.ShapeDtypeStruct((NC, N_PAD, CW), jnp.float32),
        mesh=_mesh(),
        compiler_params=pltpu.CompilerParams(use_tc_tiling_on_sc=False),
        scratch_types=[
            pltpu.VMEM((3, BPT, FB), jnp.int32),
            pltpu.VMEM((FB, CW), jnp.float32),
            pltpu.VMEM((784, CW), jnp.float32),
            pltpu.VMEM_SHARED((N_PAD, CW), jnp.float32),
        ],
    )
    def k(fidx_h, cnt_h, idx_v, ones_v, z_v, acc):
        cid = lax.axis_index("c")
        sid = lax.axis_index("s")
        wid = cid * NS + sid
        pltpu.sync_copy(fidx_h.at[:, pl.ds(wid * BPT, BPT), :], idx_v)

        one = jnp.ones((16,), jnp.float32)
        zero = jnp.zeros((16,), jnp.float32)

        def fill_ones(i, carry):
            ones_v[i, pl.ds(0, 16)] = one
            return carry

        lax.fori_loop(0, FB, fill_ones, 0)

        def fill_zero(i, carry):
            z_v[i, pl.ds(0, 16)] = zero
            return carry

        lax.fori_loop(0, 784, fill_zero, 0)

        for r in range(RPT // 784):
            pltpu.sync_copy(z_v, acc.at[pl.ds(sid * RPT + r * 784, 784), :])
        plsc.subcore_barrier()

        def body(b, carry):
            pltpu.sync_copy(ones_v, acc.at[idx_v.at[0, b]], add=True)
            pltpu.sync_copy(ones_v, acc.at[idx_v.at[1, b]], add=True)
            pltpu.sync_copy(ones_v, acc.at[idx_v.at[2, b]], add=True)
            return carry

        lax.fori_loop(0, BPT, body, 0)
        plsc.subcore_barrier()

        pltpu.sync_copy(acc.at[pl.ds(sid * RPT, RPT), :],
                        cnt_h.at[cid, pl.ds(sid * RPT, RPT), :])

    return k(fidx)


def _sc_scatter3(t, fidx, cout):
    WC = 16 if cout == 32 else 32
    n_chunks = cout // WC
    npass = (n_chunks + 1) // 2

    @functools.partial(
        pl.kernel,
        out_type=jax.ShapeDtypeStruct((N_PAD, cout), jnp.float32),
        mesh=_mesh(),
        compiler_params=pltpu.CompilerParams(use_tc_tiling_on_sc=False),
        scratch_types=[
            pltpu.VMEM((3, BPT, FB), jnp.int32),
            pltpu.VMEM((2, FB, WC), jnp.float32),
            pltpu.VMEM((196, WC), jnp.float32),
            pltpu.VMEM_SHARED((N_PAD, WC), jnp.float32),
            pltpu.SemaphoreType.DMA,
            pltpu.SemaphoreType.DMA,
            pltpu.SemaphoreType.DMA,
            pltpu.SemaphoreType.DMA,
        ],
    )
    def k(t_h, fidx_h, out_h, idx_v, tbuf, z_v, acc, l0, l1, a0, a1):
        cid = lax.axis_index("c")
        sid = lax.axis_index("s")
        lsem = (l0, l1)
        asem = (a0, a1)

        zero = jnp.zeros((16,), jnp.float32)

        def fill_zero(i, carry):
            for q in range(WC // 16):
                z_v[i, pl.ds(q * 16, 16)] = zero
            return carry

        lax.fori_loop(0, 196, fill_zero, 0)

        for jj in range(npass):
            j = jj * 2 + cid

            @pl.when(j < n_chunks)
            def _pass():
                c0 = j * WC
                for r in range(RPT // 196):
                    pltpu.sync_copy(z_v, acc.at[pl.ds(sid * RPT + r * 196, 196), :])
                plsc.subcore_barrier()

                def load(fbase, b, p):
                    pltpu.async_copy(
                        t_h.at[pl.ds(fbase + b * FB, FB), pl.ds(c0, WC)],
                        tbuf.at[p], lsem[p])

                def adds(b, p):
                    for c in range(3):
                        pltpu.async_copy(tbuf.at[p], acc.at[idx_v.at[c, b]],
                                         asem[p], add=True)

                def wait_l(p):
                    pltpu.make_async_copy(
                        t_h.at[pl.ds(0, FB), pl.ds(0, WC)], tbuf.at[p],
                        lsem[p]).wait()

                def wait_a(p):
                    for c in range(3):
                        pltpu.make_async_copy(tbuf.at[p],
                                              acc.at[pl.ds(0, FB), :],
                                              asem[p]).wait()

                for half in range(2):
                    blk0 = sid * BPS + half * BPT
                    fbase = blk0 * FB
                    pltpu.sync_copy(fidx_h.at[:, pl.ds(blk0, BPT), :], idx_v)
                    load(fbase, 0, 0)

                    def body(kk, carry, fbase=fbase):
                        u = 2 * kk
                        wait_l(0)

                        @pl.when(kk > 0)
                        def _():
                            wait_a(1)

                        load(fbase, u + 1, 1)
                        adds(u, 0)
                        wait_l(1)
                        wait_a(0)

                        @pl.when(u + 2 < BPT)
                        def _():
                            load(fbase, u + 2, 0)

                        adds(u + 1, 1)
                        return carry

                    lax.fori_loop(0, BPT // 2, body, 0)
                    wait_l(0)
                    wait_a(1)
                    adds(BPT - 1, 0)
                    wait_a(0)

                plsc.subcore_barrier()
                pltpu.sync_copy(
                    acc.at[pl.ds(sid * RPT, RPT), :],
                    out_h.at[pl.ds(sid * RPT, RPT), pl.ds(c0, WC)])
                plsc.subcore_barrier()

    return k(t, fidx)


def _tc_inv(cnt):
    def body(c_ref, o_ref):
        c = c_ref[0] + c_ref[1]
        o_ref[...] = 1.0 / jnp.maximum(c, 1.0)

    return pl.pallas_call(
        body,
        out_shape=jax.ShapeDtypeStruct((392, 128), jnp.float32),
    )(cnt)


def _tc_geom(va, vb, vc, centers_p, w0_p, wm0_p):
    BF = 512

    def body(va_ref, vb_ref, vc_ref, cen_ref, w0_ref, wm_ref, t0_ref, cf_ref):
        V1 = va_ref[:, 0:3]
        V2 = vb_ref[:, 0:3]
        V3 = vc_ref[:, 0:3]
        D12 = V2 - V1
        D23 = V3 - V2
        D31 = V1 - V3
        L12 = jnp.sqrt(jnp.sum(D12 * D12, axis=-1, keepdims=True))
        L23 = jnp.sqrt(jnp.sum(D23 * D23, axis=-1, keepdims=True))
        L31 = jnp.sqrt(jnp.sum(D31 * D31, axis=-1, keepdims=True))
        Bv = -D31
        ax, ay, az = D12[:, 0:1], D12[:, 1:2], D12[:, 2:3]
        bx, by, bz = Bv[:, 0:1], Bv[:, 1:2], Bv[:, 2:3]
        nrm = jnp.concatenate(
            [ay * bz - az * by, az * bx - ax * bz, ax * by - ay * bx], axis=-1)
        nl = jnp.sqrt(jnp.sum(nrm * nrm, axis=-1, keepdims=True))
        normals = nrm / (nl + EPS)
        T1 = jnp.sum(D12 * -D31, axis=-1, keepdims=True) / (L12 * L31 + EPS)
        T2 = jnp.sum(-D12 * D23, axis=-1, keepdims=True) / (L12 * L23 + EPS)
        T3 = jnp.sum(-D23 * D31, axis=-1, keepdims=True) / (L23 * L31 + EPS)
        zeros7 = jnp.zeros((BF, 7), jnp.float32)
        geo16 = jnp.concatenate(
            [L12, L23, L31, T1, T2, T3, normals, zeros7], axis=-1)
        n8 = jnp.concatenate([normals, jnp.zeros((BF, 5), jnp.float32)], axis=-1)
        logits = jnp.dot(n8, cen_ref[...], precision=_HIGH) * (1.0 / TAU)
        col = lax.broadcasted_iota(jnp.int32, (BF, 32), 1)
        logits = jnp.where(col < K, logits, -1e30)
        m = jnp.max(logits, axis=-1, keepdims=True)
        e = jnp.exp(logits - m)
        coeff = e / jnp.sum(e, axis=-1, keepdims=True)
        hf0 = jnp.maximum(jnp.dot(geo16, w0_ref[...], precision=_HIGH), 0.0)
        scale = jnp.dot(coeff, wm_ref[...], precision=_HIGH)
        t0_ref[...] = hf0 * scale
        cf_ref[...] = coeff

    grid = (F_PAD // BF,)
    return pl.pallas_call(
        body,
        grid=grid,
        in_specs=[
            pl.BlockSpec((BF, 8), lambda i: (i, 0)),
            pl.BlockSpec((BF, 8), lambda i: (i, 0)),
            pl.BlockSpec((BF, 8), lambda i: (i, 0)),
            pl.BlockSpec((8, 32), lambda i: (0, 0)),
            pl.BlockSpec((16, 32), lambda i: (0, 0)),
            pl.BlockSpec((32, 32), lambda i: (0, 0)),
        ],
        out_specs=[
            pl.BlockSpec((BF, 32), lambda i: (i, 0)),
            pl.BlockSpec((BF, 32), lambda i: (i, 0)),
        ],
        out_shape=[
            jax.ShapeDtypeStruct((F_PAD, 32), jnp.float32),
            jax.ShapeDtypeStruct((F_PAD, 32), jnp.float32),
        ],
    )(va, vb, vc, centers_p, w0_p, wm0_p)


def _tc_layer(ga, gb, gc, coeff, wv_s, wm_p):
    BF = 256
    cin = ga.shape[1]
    cout = wv_s.shape[1]

    def body(ga_ref, gb_ref, gc_ref, cf_ref, wv_ref, wm_ref, t_ref):
        g = ga_ref[...] + gb_ref[...] + gc_ref[...]
        hf = jnp.maximum(jnp.dot(g, wv_ref[...], precision=_HIGH), 0.0)
        scale = jnp.dot(cf_ref[...], wm_ref[...], precision=_HIGH)
        t_ref[...] = hf * scale

    return pl.pallas_call(
        body,
        grid=(F_PAD // BF,),
        in_specs=[
            pl.BlockSpec((BF, cin), lambda i: (i, 0)),
            pl.BlockSpec((BF, cin), lambda i: (i, 0)),
            pl.BlockSpec((BF, cin), lambda i: (i, 0)),
            pl.BlockSpec((BF, 32), lambda i: (i, 0)),
            pl.BlockSpec((cin, cout), lambda i: (0, 0)),
            pl.BlockSpec((32, cout), lambda i: (0, 0)),
        ],
        out_specs=pl.BlockSpec((BF, cout), lambda i: (i, 0)),
        out_shape=jax.ShapeDtypeStruct((F_PAD, cout), jnp.float32),
    )(ga, gb, gc, coeff, wv_s, wm_p)


def _tc_feats(acc, inv):
    BV = 512
    c = acc.shape[1]

    def body(a_ref, i_ref, o_ref):
        o_ref[...] = jnp.maximum(a_ref[...], 0.0) * i_ref[...]

    return pl.pallas_call(
        body,
        grid=(N_PAD // BV,),
        in_specs=[
            pl.BlockSpec((BV, c), lambda i: (i, 0)),
            pl.BlockSpec((BV, 1), lambda i: (i, 0)),
        ],
        out_specs=pl.BlockSpec((BV, c), lambda i: (i, 0)),
        out_shape=jax.ShapeDtypeStruct((N_PAD, c), jnp.float32),
    )(acc, inv)


def _tc_pool(feats3):
    seg = N // B

    def body(f_ref, o_ref):
        o_ref[0, 0, :] = jnp.sum(f_ref[0], axis=0) * (1.0 / seg)

    return pl.pallas_call(
        body,
        grid=(B,),
        in_specs=[pl.BlockSpec((1, seg, 256), lambda i: (i, 0, 0))],
        out_specs=pl.BlockSpec((1, 1, 256), lambda i: (i, 0, 0)),
        out_shape=jax.ShapeDtypeStruct((B, 1, 256), jnp.float32),
    )(feats3)


def _tc_fc(pooled, w1, b1, w2, b2):
    def body(p_ref, w1_ref, b1_ref, w2_ref, b2_ref, o_ref):
        h = jnp.maximum(
            jnp.dot(p_ref[...], w1_ref[...], precision=_HIGH) + b1_ref[...], 0.0)
        o_ref[...] = jnp.dot(h, w2_ref[...], precision=_HIGH) + b2_ref[...]

    return pl.pallas_call(
        body,
        out_shape=jax.ShapeDtypeStruct((B, 20), jnp.float32),
    )(pooled, w1, b1, w2, b2)


def kernel(vertex_in, face_in, nv_in, mf_in, centers, W0, Wm0, Wv0, Wv1, Wv2,
           Wv3, Wv4, Wmb0, Wmb1, Wmb2, Wmb3, Wmb4, W_fc1, b_fc1, W_fc2, b_fc2):
    face32 = face_in.astype(jnp.int32)
    pad_idx = jnp.full((F_PAD - F, 3), JUNK, jnp.int32)
    fidx = jnp.concatenate([face32, pad_idx], axis=0).T.reshape(3, NBLK, FB)
    vert_p = jnp.pad(vertex_in, ((0, N_PAD - N), (0, 5)))

    centers_p = jnp.zeros((8, 32), jnp.float32).at[0:3, 0:K].set(centers.T)
    w0_p = jnp.pad(W0, ((0, 7), (0, 0)))
    wm0_p = jnp.pad(Wm0, ((0, 32 - K), (0, 0)))
    wvs = [w / 3.0 for w in (Wv0, Wv1, Wv2, Wv3, Wv4)]
    wms = [jnp.pad(w, ((0, 32 - K), (0, 0))) for w in (Wmb0, Wmb1, Wmb2, Wmb3, Wmb4)]

    cnt = _sc_count(fidx)[:, :, 0]
    inv = _tc_inv(cnt.reshape(2, 392, 128)).reshape(N_PAD, 1)

    g3v = _sc_gather3(vert_p, fidx)
    t0, coeff = _tc_geom(g3v[0], g3v[1], g3v[2], centers_p, w0_p, wm0_p)

    acc = _sc_scatter3(t0, fidx, CH[0])
    feats = _tc_feats(acc, inv)
    for l in range(5):
        g3 = _sc_gather3(feats, fidx)
        t = _tc_layer(g3[0], g3[1], g3[2], coeff, wvs[l], wms[l])
        acc = _sc_scatter3(t, fidx, CH[l + 1])
        feats = _tc_feats(acc, inv)

    pooled = _tc_pool(feats[:N].reshape(B, N // B, 256)).reshape(B, 256)
    logits = _tc_fc(pooled, W_fc1, b_fc1.reshape(1, 128), W_fc2,
                    b_fc2.reshape(1, 20))
    return logits


# speedup vs baseline: 1.7232x; 1.0943x over previous
import functools

import jax
import jax.numpy as jnp
from jax import lax
from jax.experimental import pallas as pl
from jax.experimental.pallas import tpu as pltpu
from jax.experimental.pallas import tpu_sc as plsc

N = 50000
F = 100000
B = 8
K = 27
CH = [32, 64, 96, 128, 192, 256]
TAU = 0.1
EPS = 1e-8

NC = 2
NS = 16
NW = NC * NS

FB = 128
N_PAD = 50176
F_PAD = 102400
NBLK = F_PAD // FB
BPT = NBLK // NW
BPS = NBLK // NS
RPT = N_PAD // NS
JUNK = N_PAD - 1

_HIGH = lax.Precision.HIGHEST


def _mesh():
    return plsc.VectorSubcoreMesh(core_axis_name="c", subcore_axis_name="s")


def _sc_gather3(table, fidx):
    D = table.shape[1]

    SPB = 1 if D <= 128 else 2
    FBG = FB // SPB
    U = BPT * SPB
    KB = (U + 1) // 2

    @functools.partial(
        pl.kernel,
        out_type=jax.ShapeDtypeStruct((3, F_PAD, D), jnp.float32),
        mesh=_mesh(),
        compiler_params=pltpu.CompilerParams(use_tc_tiling_on_sc=False),
        scratch_types=[
            pltpu.VMEM((3, BPT, FB), jnp.int32),
            pltpu.VMEM((2, 3, FBG, D), jnp.float32),
            pltpu.SemaphoreType.DMA,
            pltpu.SemaphoreType.DMA,
            pltpu.SemaphoreType.DMA,
            pltpu.SemaphoreType.DMA,
        ],
    )
    def k(table_h, fidx_h, out_h, idx_v, buf, g0, g1, w0, w1):
        cid = lax.axis_index("c")
        sid = lax.axis_index("s")
        wid = cid * NS + sid
        pltpu.sync_copy(fidx_h.at[:, pl.ds(wid * BPT, BPT), :], idx_v)
        fbase = wid * BPT * FB
        gsem = (g0, g1)
        wsem = (w0, w1)

        def idx_at(c, b, h):
            if SPB == 1:
                return idx_v.at[c, b]
            return idx_v.at[c, b, pl.ds(h * FBG, FBG)]

        def gather(u, b, h, p):
            for c in range(3):
                pltpu.async_copy(table_h.at[idx_at(c, b, h)], buf.at[p, c],
                                 gsem[p])

        def writeback(u, p):
            row0 = fbase + u * FBG
            for c in range(3):
                pltpu.async_copy(buf.at[p, c], out_h.at[c, pl.ds(row0, FBG), :],
                                 wsem[p])

        def wait_g(p):
            for c in range(3):
                pltpu.make_async_copy(table_h.at[pl.ds(0, FBG)], buf.at[p, c],
                                      gsem[p]).wait()

        def wait_w(p):
            for c in range(3):
                pltpu.make_async_copy(buf.at[p, c],
                                      out_h.at[c, pl.ds(0, FBG), :],
                                      wsem[p]).wait()

        gather(0, 0, 0, 0)

        def body(kk, carry):
            u = 2 * kk
            wait_g(0)

            @pl.when(kk > 0)
            def _():
                wait_w(1)

            @pl.when(u + 1 < U)
            def _():
                b1, h1 = (u + 1, 0) if SPB == 1 else (kk, 1)
                gather(u + 1, b1, h1, 1)

            writeback(u, 0)

            @pl.when(u + 1 < U)
            def _():
                wait_g(1)
                wait_w(0)

                @pl.when(u + 2 < U)
                def _():
                    b2, h2 = (u + 2, 0) if SPB == 1 else (kk + 1, 0)
                    gather(u + 2, b2, h2, 0)

                writeback(u + 1, 1)

            return carry

        lax.fori_loop(0, KB, body, 0)
        if U % 2 == 1:
            wait_w(0)
        else:
            wait_w(1)

    return k(table, fidx)


def _sc_count(fidx):
    CW = 16

    @functools.partial(
        pl.kernel,
        out_type=jax.ShapeDtypeStruct((NC, N_PAD, CW), jnp.float32),
        mesh=_mesh(),
        compiler_params=pltpu.CompilerParams(use_tc_tiling_on_sc=False),
        scratch_types=[
            pltpu.VMEM((3, BPT, FB), jnp.int32),
            pltpu.VMEM((FB, CW), jnp.float32),
            pltpu.VMEM((784, CW), jnp.float32),
            pltpu.VMEM_SHARED((N_PAD, CW), jnp.float32),
        ],
    )
    def k(fidx_h, cnt_h, idx_v, ones_v, z_v, acc):
        cid = lax.axis_index("c")
        sid = lax.axis_index("s")
        wid = cid * NS + sid
        pltpu.sync_copy(fidx_h.at[:, pl.ds(wid * BPT, BPT), :], idx_v)

        one = jnp.ones((16,), jnp.float32)
        zero = jnp.zeros((16,), jnp.float32)

        def fill_ones(i, carry):
            ones_v[i, pl.ds(0, 16)] = one
            return carry

        lax.fori_loop(0, FB, fill_ones, 0)

        def fill_zero(i, carry):
            z_v[i, pl.ds(0, 16)] = zero
            return carry

        lax.fori_loop(0, 784, fill_zero, 0)

        for r in range(RPT // 784):
            pltpu.sync_copy(z_v, acc.at[pl.ds(sid * RPT + r * 784, 784), :])
        plsc.subcore_barrier()

        def body(b, carry):
            pltpu.sync_copy(ones_v, acc.at[idx_v.at[0, b]], add=True)
            pltpu.sync_copy(ones_v, acc.at[idx_v.at[1, b]], add=True)
            pltpu.sync_copy(ones_v, acc.at[idx_v.at[2, b]], add=True)
            return carry

        lax.fori_loop(0, BPT, body, 0)
        plsc.subcore_barrier()

        pltpu.sync_copy(acc.at[pl.ds(sid * RPT, RPT), :],
                        cnt_h.at[cid, pl.ds(sid * RPT, RPT), :])

    return k(fidx)


def _sc_scatter3(t, fidx, inv, cout):
    WC = 16 if cout == 32 else 32
    n_chunks = cout // WC
    npass = (n_chunks + 1) // 2

    @functools.partial(
        pl.kernel,
        out_type=jax.ShapeDtypeStruct((N_PAD, cout), jnp.float32),
        mesh=_mesh(),
        compiler_params=pltpu.CompilerParams(use_tc_tiling_on_sc=False),
        scratch_types=[
            pltpu.VMEM((3, BPT, FB), jnp.int32),
            pltpu.VMEM((2, FB, WC), jnp.float32),
            pltpu.VMEM((224, WC), jnp.float32),
            pltpu.VMEM((RPT,), jnp.float32),
            pltpu.VMEM_SHARED((N_PAD, WC), jnp.float32),
            pltpu.SemaphoreType.DMA,
            pltpu.SemaphoreType.DMA,
            pltpu.SemaphoreType.DMA,
            pltpu.SemaphoreType.DMA,
        ],
    )
    def k(t_h, fidx_h, inv_h, out_h, idx_v, tbuf, z_v, inv_v, acc,
          l0, l1, a0, a1):
        cid = lax.axis_index("c")
        sid = lax.axis_index("s")
        lsem = (l0, l1)
        asem = (a0, a1)
        pltpu.sync_copy(inv_h.at[pl.ds(sid * RPT, RPT)], inv_v)

        zero = jnp.zeros((16,), jnp.float32)

        def fill_zero(i, carry):
            for q in range(WC // 16):
                z_v[i, pl.ds(q * 16, 16)] = zero
            return carry

        NZ = 224
        for jj in range(npass):
            j = jj * 2 + cid

            @pl.when(j < n_chunks)
            def _pass():
                c0 = j * WC
                lax.fori_loop(0, NZ, fill_zero, 0)
                for r in range(RPT // NZ):
                    pltpu.sync_copy(z_v, acc.at[pl.ds(sid * RPT + r * NZ, NZ), :])
                plsc.subcore_barrier()

                def load(fbase, b, p):
                    pltpu.async_copy(
                        t_h.at[pl.ds(fbase + b * FB, FB), pl.ds(c0, WC)],
                        tbuf.at[p], lsem[p])

                def adds(b, p):
                    for c in range(3):
                        pltpu.async_copy(tbuf.at[p], acc.at[idx_v.at[c, b]],
                                         asem[p], add=True)

                def wait_l(p):
                    pltpu.make_async_copy(
                        t_h.at[pl.ds(0, FB), pl.ds(0, WC)], tbuf.at[p],
                        lsem[p]).wait()

                def wait_a(p):
                    for c in range(3):
                        pltpu.make_async_copy(tbuf.at[p],
                                              acc.at[pl.ds(0, FB), :],
                                              asem[p]).wait()

                for half in range(2):
                    blk0 = sid * BPS + half * BPT
                    fbase = blk0 * FB
                    pltpu.sync_copy(fidx_h.at[:, pl.ds(blk0, BPT), :], idx_v)
                    load(fbase, 0, 0)

                    def body(kk, carry, fbase=fbase):
                        u = 2 * kk
                        wait_l(0)

                        @pl.when(kk > 0)
                        def _():
                            wait_a(1)

                        load(fbase, u + 1, 1)
                        adds(u, 0)
                        wait_l(1)
                        wait_a(0)

                        @pl.when(u + 2 < BPT)
                        def _():
                            load(fbase, u + 2, 0)

                        adds(u + 1, 1)
                        return carry

                    lax.fori_loop(0, BPT // 2, body, 0)
                    wait_l(0)
                    wait_a(1)
                    adds(BPT - 1, 0)
                    wait_a(0)

                plsc.subcore_barrier()

                def wb_round(r, carry):
                    row0 = sid * RPT + r * NZ
                    pltpu.sync_copy(acc.at[pl.ds(row0, NZ), :], z_v)

                    def scale_group(g, carry2):
                        iv16 = inv_v[pl.ds(r * NZ + g * 16, 16)]
                        for q in range(16):
                            iv = iv16[q]
                            for qc in range(WC // 16):
                                x = z_v[g * 16 + q, pl.ds(qc * 16, 16)]
                                z_v[g * 16 + q, pl.ds(qc * 16, 16)] = (
                                    jnp.maximum(x, 0.0) * iv)
                        return carry2

                    lax.fori_loop(0, NZ // 16, scale_group, 0)
                    pltpu.sync_copy(
                        z_v, out_h.at[pl.ds(row0, NZ), pl.ds(c0, WC)])
                    return carry

                lax.fori_loop(0, RPT // NZ, wb_round, 0)
                plsc.subcore_barrier()

    return k(t, fidx, inv)


def _tc_inv(cnt):
    def body(c_ref, o_ref):
        c = c_ref[0] + c_ref[1]
        o_ref[...] = 1.0 / jnp.maximum(c, 1.0)

    return pl.pallas_call(
        body,
        out_shape=jax.ShapeDtypeStruct((392, 128), jnp.float32),
    )(cnt)


def _tc_geom(va, vb, vc, centers_p, w0_p, wm0_p):
    BF = 512

    def body(va_ref, vb_ref, vc_ref, cen_ref, w0_ref, wm_ref, t0_ref, cf_ref):
        V1 = va_ref[:, 0:3]
        V2 = vb_ref[:, 0:3]
        V3 = vc_ref[:, 0:3]
        D12 = V2 - V1
        D23 = V3 - V2
        D31 = V1 - V3
        L12 = jnp.sqrt(jnp.sum(D12 * D12, axis=-1, keepdims=True))
        L23 = jnp.sqrt(jnp.sum(D23 * D23, axis=-1, keepdims=True))
        L31 = jnp.sqrt(jnp.sum(D31 * D31, axis=-1, keepdims=True))
        Bv = -D31
        ax, ay, az = D12[:, 0:1], D12[:, 1:2], D12[:, 2:3]
        bx, by, bz = Bv[:, 0:1], Bv[:, 1:2], Bv[:, 2:3]
        nrm = jnp.concatenate(
            [ay * bz - az * by, az * bx - ax * bz, ax * by - ay * bx], axis=-1)
        nl = jnp.sqrt(jnp.sum(nrm * nrm, axis=-1, keepdims=True))
        normals = nrm / (nl + EPS)
        T1 = jnp.sum(D12 * -D31, axis=-1, keepdims=True) / (L12 * L31 + EPS)
        T2 = jnp.sum(-D12 * D23, axis=-1, keepdims=True) / (L12 * L23 + EPS)
        T3 = jnp.sum(-D23 * D31, axis=-1, keepdims=True) / (L23 * L31 + EPS)
        zeros7 = jnp.zeros((BF, 7), jnp.float32)
        geo16 = jnp.concatenate(
            [L12, L23, L31, T1, T2, T3, normals, zeros7], axis=-1)
        n8 = jnp.concatenate([normals, jnp.zeros((BF, 5), jnp.float32)], axis=-1)
        logits = jnp.dot(n8, cen_ref[...], precision=_HIGH) * (1.0 / TAU)
        col = lax.broadcasted_iota(jnp.int32, (BF, 32), 1)
        logits = jnp.where(col < K, logits, -1e30)
        m = jnp.max(logits, axis=-1, keepdims=True)
        e = jnp.exp(logits - m)
        coeff = e / jnp.sum(e, axis=-1, keepdims=True)
        hf0 = jnp.maximum(jnp.dot(geo16, w0_ref[...], precision=_HIGH), 0.0)
        scale = jnp.dot(coeff, wm_ref[...], precision=_HIGH)
        t0_ref[...] = hf0 * scale
        cf_ref[...] = coeff

    grid = (F_PAD // BF,)
    return pl.pallas_call(
        body,
        grid=grid,
        in_specs=[
            pl.BlockSpec((BF, 8), lambda i: (i, 0)),
            pl.BlockSpec((BF, 8), lambda i: (i, 0)),
            pl.BlockSpec((BF, 8), lambda i: (i, 0)),
            pl.BlockSpec((8, 32), lambda i: (0, 0)),
            pl.BlockSpec((16, 32), lambda i: (0, 0)),
            pl.BlockSpec((32, 32), lambda i: (0, 0)),
        ],
        out_specs=[
            pl.BlockSpec((BF, 32), lambda i: (i, 0)),
            pl.BlockSpec((BF, 32), lambda i: (i, 0)),
        ],
        out_shape=[
            jax.ShapeDtypeStruct((F_PAD, 32), jnp.float32),
            jax.ShapeDtypeStruct((F_PAD, 32), jnp.float32),
        ],
    )(va, vb, vc, centers_p, w0_p, wm0_p)


def _tc_layer(ga, gb, gc, coeff, wv_s, wm_p):
    BF = 256
    cin = ga.shape[1]
    cout = wv_s.shape[1]

    def body(ga_ref, gb_ref, gc_ref, cf_ref, wv_ref, wm_ref, t_ref):
        g = ga_ref[...] + gb_ref[...] + gc_ref[...]
        hf = jnp.maximum(jnp.dot(g, wv_ref[...], precision=_HIGH), 0.0)
        scale = jnp.dot(cf_ref[...], wm_ref[...], precision=_HIGH)
        t_ref[...] = hf * scale

    return pl.pallas_call(
        body,
        grid=(F_PAD // BF,),
        in_specs=[
            pl.BlockSpec((BF, cin), lambda i: (i, 0)),
            pl.BlockSpec((BF, cin), lambda i: (i, 0)),
            pl.BlockSpec((BF, cin), lambda i: (i, 0)),
            pl.BlockSpec((BF, 32), lambda i: (i, 0)),
            pl.BlockSpec((cin, cout), lambda i: (0, 0)),
            pl.BlockSpec((32, cout), lambda i: (0, 0)),
        ],
        out_specs=pl.BlockSpec((BF, cout), lambda i: (i, 0)),
        out_shape=jax.ShapeDtypeStruct((F_PAD, cout), jnp.float32),
    )(ga, gb, gc, coeff, wv_s, wm_p)


def _tc_pool(feats3):
    seg = N // B

    def body(f_ref, o_ref):
        o_ref[0, 0, :] = jnp.sum(f_ref[0], axis=0) * (1.0 / seg)

    return pl.pallas_call(
        body,
        grid=(B,),
        in_specs=[pl.BlockSpec((1, seg, 256), lambda i: (i, 0, 0))],
        out_specs=pl.BlockSpec((1, 1, 256), lambda i: (i, 0, 0)),
        out_shape=jax.ShapeDtypeStruct((B, 1, 256), jnp.float32),
    )(feats3)


def _tc_fc(pooled, w1, b1, w2, b2):
    def body(p_ref, w1_ref, b1_ref, w2_ref, b2_ref, o_ref):
        h = jnp.maximum(
            jnp.dot(p_ref[...], w1_ref[...], precision=_HIGH) + b1_ref[...], 0.0)
        o_ref[...] = jnp.dot(h, w2_ref[...], precision=_HIGH) + b2_ref[...]

    return pl.pallas_call(
        body,
        out_shape=jax.ShapeDtypeStruct((B, 20), jnp.float32),
    )(pooled, w1, b1, w2, b2)


def kernel(vertex_in, face_in, nv_in, mf_in, centers, W0, Wm0, Wv0, Wv1, Wv2,
           Wv3, Wv4, Wmb0, Wmb1, Wmb2, Wmb3, Wmb4, W_fc1, b_fc1, W_fc2, b_fc2):
    face32 = face_in.astype(jnp.int32)
    pad_idx = jnp.full((F_PAD - F, 3), JUNK, jnp.int32)
    fidx = jnp.concatenate([face32, pad_idx], axis=0).T.reshape(3, NBLK, FB)
    vert_p = jnp.pad(vertex_in, ((0, N_PAD - N), (0, 5)))

    centers_p = jnp.zeros((8, 32), jnp.float32).at[0:3, 0:K].set(centers.T)
    w0_p = jnp.pad(W0, ((0, 7), (0, 0)))
    wm0_p = jnp.pad(Wm0, ((0, 32 - K), (0, 0)))
    wvs = [w / 3.0 for w in (Wv0, Wv1, Wv2, Wv3, Wv4)]
    wms = [jnp.pad(w, ((0, 32 - K), (0, 0))) for w in (Wmb0, Wmb1, Wmb2, Wmb3, Wmb4)]

    cnt = _sc_count(fidx)[:, :, 0]
    inv = _tc_inv(cnt.reshape(2, 392, 128)).reshape(N_PAD)

    g3v = _sc_gather3(vert_p, fidx)
    t0, coeff = _tc_geom(g3v[0], g3v[1], g3v[2], centers_p, w0_p, wm0_p)

    feats = _sc_scatter3(t0, fidx, inv, CH[0])
    for l in range(5):
        g3 = _sc_gather3(feats, fidx)
        t = _tc_layer(g3[0], g3[1], g3[2], coeff, wvs[l], wms[l])
        feats = _sc_scatter3(t, fidx, inv, CH[l + 1])

    pooled = _tc_pool(feats[:N].reshape(B, N // B, 256)).reshape(B, 256)
    logits = _tc_fc(pooled, W_fc1, b_fc1.reshape(1, 128), W_fc2,
                    b_fc2.reshape(1, 20))
    return logits


# speedup vs baseline: 1.7855x; 1.0362x over previous
import functools

import jax
import jax.numpy as jnp
from jax import lax
from jax.experimental import pallas as pl
from jax.experimental.pallas import tpu as pltpu
from jax.experimental.pallas import tpu_sc as plsc

N = 50000
F = 100000
B = 8
K = 27
CH = [32, 64, 96, 128, 192, 256]
TAU = 0.1
EPS = 1e-8

NC = 2
NS = 16
NW = NC * NS

FB = 128
N_PAD = 50176
F_PAD = 102400
NBLK = F_PAD // FB
BPT = NBLK // NW
BPS = NBLK // NS
RPT = N_PAD // NS
JUNK = N_PAD - 1

_HIGH = lax.Precision.HIGHEST


def _mesh():
    return plsc.VectorSubcoreMesh(core_axis_name="c", subcore_axis_name="s")


def _sc_gather3(table, fidx, with_count=False):
    D = table.shape[1]

    SPB = 1 if D <= 128 else 2
    FBG = FB // SPB
    U = BPT * SPB
    KB = (U + 1) // 2
    CW = 16

    out_type = [jax.ShapeDtypeStruct((3, F_PAD, D), jnp.float32)]
    scratch = [
        pltpu.VMEM((3, BPT, FB), jnp.int32),
        pltpu.VMEM((2, 3, FBG, D), jnp.float32),
        pltpu.SemaphoreType.DMA,
        pltpu.SemaphoreType.DMA,
        pltpu.SemaphoreType.DMA,
        pltpu.SemaphoreType.DMA,
    ]
    if with_count:
        out_type.append(jax.ShapeDtypeStruct((NC, N_PAD, CW), jnp.float32))
        scratch += [
            pltpu.VMEM((FB, CW), jnp.float32),
            pltpu.VMEM((224, CW), jnp.float32),
            pltpu.VMEM_SHARED((N_PAD, CW), jnp.float32),
            pltpu.SemaphoreType.DMA,
        ]

    @functools.partial(
        pl.kernel,
        out_type=tuple(out_type),
        mesh=_mesh(),
        compiler_params=pltpu.CompilerParams(use_tc_tiling_on_sc=False),
        scratch_types=scratch,
    )
    def k(table_h, fidx_h, out_h, *rest):
        if with_count:
            (cnt_h, idx_v, buf, g0, g1, w0, w1, ones_v, z_v, cacc, csem) = rest
        else:
            (idx_v, buf, g0, g1, w0, w1) = rest
        cid = lax.axis_index("c")
        sid = lax.axis_index("s")
        wid = cid * NS + sid
        pltpu.sync_copy(fidx_h.at[:, pl.ds(wid * BPT, BPT), :], idx_v)
        fbase = wid * BPT * FB
        gsem = (g0, g1)
        wsem = (w0, w1)

        if with_count:
            one = jnp.ones((16,), jnp.float32)
            zero = jnp.zeros((16,), jnp.float32)

            def fill_ones(i, carry):
                ones_v[i, pl.ds(0, 16)] = one
                return carry

            lax.fori_loop(0, FB, fill_ones, 0)

            def fill_zero(i, carry):
                z_v[i, pl.ds(0, 16)] = zero
                return carry

            lax.fori_loop(0, 224, fill_zero, 0)
            for r in range(RPT // 224):
                pltpu.sync_copy(z_v, cacc.at[pl.ds(sid * RPT + r * 224, 224), :])
            plsc.subcore_barrier()

        def idx_at(c, b, h):
            if SPB == 1:
                return idx_v.at[c, b]
            return idx_v.at[c, b, pl.ds(h * FBG, FBG)]

        def gather(u, b, h, p):
            for c in range(3):
                pltpu.async_copy(table_h.at[idx_at(c, b, h)], buf.at[p, c],
                                 gsem[p])
                if with_count:
                    pltpu.async_copy(ones_v.at[pl.ds(0, FBG)],
                                     cacc.at[idx_at(c, b, h)], csem, add=True)

        def writeback(u, p):
            row0 = fbase + u * FBG
            for c in range(3):
                pltpu.async_copy(buf.at[p, c], out_h.at[c, pl.ds(row0, FBG), :],
                                 wsem[p])

        def wait_g(p):
            for c in range(3):
                pltpu.make_async_copy(table_h.at[pl.ds(0, FBG)], buf.at[p, c],
                                      gsem[p]).wait()

        def wait_w(p):
            for c in range(3):
                pltpu.make_async_copy(buf.at[p, c],
                                      out_h.at[c, pl.ds(0, FBG), :],
                                      wsem[p]).wait()

        gather(0, 0, 0, 0)

        def body(kk, carry):
            u = 2 * kk
            wait_g(0)

            @pl.when(kk > 0)
            def _():
                wait_w(1)

            @pl.when(u + 1 < U)
            def _():
                b1, h1 = (u + 1, 0) if SPB == 1 else (kk, 1)
                gather(u + 1, b1, h1, 1)

            writeback(u, 0)

            @pl.when(u + 1 < U)
            def _():
                wait_g(1)
                wait_w(0)

                @pl.when(u + 2 < U)
                def _():
                    b2, h2 = (u + 2, 0) if SPB == 1 else (kk + 1, 0)
                    gather(u + 2, b2, h2, 0)

                writeback(u + 1, 1)

            return carry

        lax.fori_loop(0, KB, body, 0)
        if U % 2 == 1:
            wait_w(0)
        else:
            wait_w(1)

        if with_count:
            def drain(i, carry):
                for c in range(3):
                    pltpu.make_async_copy(ones_v.at[pl.ds(0, FBG)],
                                          cacc.at[pl.ds(0, FBG), :],
                                          csem).wait()
                return carry

            lax.fori_loop(0, U, drain, 0)
            plsc.subcore_barrier()
            pltpu.sync_copy(cacc.at[pl.ds(sid * RPT, RPT), :],
                            cnt_h.at[cid, pl.ds(sid * RPT, RPT), :])

    res = k(table, fidx)
    return res if with_count else res[0]


def _sc_scatter3(t, fidx, inv, cout):
    WC = 16 if cout == 32 else 32
    n_chunks = cout // WC
    npass = (n_chunks + 1) // 2

    @functools.partial(
        pl.kernel,
        out_type=jax.ShapeDtypeStruct((N_PAD, cout), jnp.float32),
        mesh=_mesh(),
        compiler_params=pltpu.CompilerParams(use_tc_tiling_on_sc=False),
        scratch_types=[
            pltpu.VMEM((3, BPT, FB), jnp.int32),
            pltpu.VMEM((2, FB, WC), jnp.float32),
            pltpu.VMEM((224, WC), jnp.float32),
            pltpu.VMEM((RPT,), jnp.float32),
            pltpu.VMEM_SHARED((N_PAD, WC), jnp.float32),
            pltpu.SemaphoreType.DMA,
            pltpu.SemaphoreType.DMA,
            pltpu.SemaphoreType.DMA,
            pltpu.SemaphoreType.DMA,
        ],
    )
    def k(t_h, fidx_h, inv_h, out_h, idx_v, tbuf, z_v, inv_v, acc,
          l0, l1, a0, a1):
        cid = lax.axis_index("c")
        sid = lax.axis_index("s")
        lsem = (l0, l1)
        asem = (a0, a1)
        pltpu.sync_copy(inv_h.at[pl.ds(sid * RPT, RPT)], inv_v)

        zero = jnp.zeros((16,), jnp.float32)

        def fill_zero(i, carry):
            for q in range(WC // 16):
                z_v[i, pl.ds(q * 16, 16)] = zero
            return carry

        NZ = 224
        for jj in range(npass):
            j = jj * 2 + cid

            @pl.when(j < n_chunks)
            def _pass():
                c0 = j * WC
                lax.fori_loop(0, NZ, fill_zero, 0)
                for r in range(RPT // NZ):
                    pltpu.sync_copy(z_v, acc.at[pl.ds(sid * RPT + r * NZ, NZ), :])
                plsc.subcore_barrier()

                def load(fbase, b, p):
                    pltpu.async_copy(
                        t_h.at[pl.ds(fbase + b * FB, FB), pl.ds(c0, WC)],
                        tbuf.at[p], lsem[p])

                def adds(b, p):
                    for c in range(3):
                        pltpu.async_copy(tbuf.at[p], acc.at[idx_v.at[c, b]],
                                         asem[p], add=True)

                def wait_l(p):
                    pltpu.make_async_copy(
                        t_h.at[pl.ds(0, FB), pl.ds(0, WC)], tbuf.at[p],
                        lsem[p]).wait()

                def wait_a(p):
                    for c in range(3):
                        pltpu.make_async_copy(tbuf.at[p],
                                              acc.at[pl.ds(0, FB), :],
                                              asem[p]).wait()

                for half in range(2):
                    blk0 = sid * BPS + half * BPT
                    fbase = blk0 * FB
                    pltpu.sync_copy(fidx_h.at[:, pl.ds(blk0, BPT), :], idx_v)
                    load(fbase, 0, 0)

                    def body(kk, carry, fbase=fbase):
                        u = 2 * kk
                        wait_l(0)

                        @pl.when(kk > 0)
                        def _():
                            wait_a(1)

                        load(fbase, u + 1, 1)
                        adds(u, 0)
                        wait_l(1)
                        wait_a(0)

                        @pl.when(u + 2 < BPT)
                        def _():
                            load(fbase, u + 2, 0)

                        adds(u + 1, 1)
                        return carry

                    lax.fori_loop(0, BPT // 2, body, 0)
                    wait_l(0)
                    wait_a(1)
                    adds(BPT - 1, 0)
                    wait_a(0)

                plsc.subcore_barrier()

                def wb_round(r, carry):
                    row0 = sid * RPT + r * NZ
                    pltpu.sync_copy(acc.at[pl.ds(row0, NZ), :], z_v)

                    def scale_group(g, carry2):
                        iv16 = inv_v[pl.ds(r * NZ + g * 16, 16)]
                        for q in range(16):
                            iv = iv16[q]
                            for qc in range(WC // 16):
                                x = z_v[g * 16 + q, pl.ds(qc * 16, 16)]
                                z_v[g * 16 + q, pl.ds(qc * 16, 16)] = (
                                    jnp.maximum(x, 0.0) * iv)
                        return carry2

                    lax.fori_loop(0, NZ // 16, scale_group, 0)
                    pltpu.sync_copy(
                        z_v, out_h.at[pl.ds(row0, NZ), pl.ds(c0, WC)])
                    return carry

                lax.fori_loop(0, RPT // NZ, wb_round, 0)
                plsc.subcore_barrier()

    return k(t, fidx, inv)


def _tc_inv(cnt):
    def body(c_ref, o_ref):
        c = c_ref[0] + c_ref[1]
        o_ref[...] = 1.0 / jnp.maximum(c, 1.0)

    return pl.pallas_call(
        body,
        out_shape=jax.ShapeDtypeStruct((392, 128), jnp.float32),
    )(cnt)


def _tc_geom(va, vb, vc, centers_p, w0_p, wm0_p):
    BF = 512

    def body(va_ref, vb_ref, vc_ref, cen_ref, w0_ref, wm_ref, t0_ref, cf_ref):
        V1 = va_ref[:, 0:3]
        V2 = vb_ref[:, 0:3]
        V3 = vc_ref[:, 0:3]
        D12 = V2 - V1
        D23 = V3 - V2
        D31 = V1 - V3
        L12 = jnp.sqrt(jnp.sum(D12 * D12, axis=-1, keepdims=True))
        L23 = jnp.sqrt(jnp.sum(D23 * D23, axis=-1, keepdims=True))
        L31 = jnp.sqrt(jnp.sum(D31 * D31, axis=-1, keepdims=True))
        Bv = -D31
        ax, ay, az = D12[:, 0:1], D12[:, 1:2], D12[:, 2:3]
        bx, by, bz = Bv[:, 0:1], Bv[:, 1:2], Bv[:, 2:3]
        nrm = jnp.concatenate(
            [ay * bz - az * by, az * bx - ax * bz, ax * by - ay * bx], axis=-1)
        nl = jnp.sqrt(jnp.sum(nrm * nrm, axis=-1, keepdims=True))
        normals = nrm / (nl + EPS)
        T1 = jnp.sum(D12 * -D31, axis=-1, keepdims=True) / (L12 * L31 + EPS)
        T2 = jnp.sum(-D12 * D23, axis=-1, keepdims=True) / (L12 * L23 + EPS)
        T3 = jnp.sum(-D23 * D31, axis=-1, keepdims=True) / (L23 * L31 + EPS)
        zeros7 = jnp.zeros((BF, 7), jnp.float32)
        geo16 = jnp.concatenate(
            [L12, L23, L31, T1, T2, T3, normals, zeros7], axis=-1)
        n8 = jnp.concatenate([normals, jnp.zeros((BF, 5), jnp.float32)], axis=-1)
        logits = jnp.dot(n8, cen_ref[...], precision=_HIGH) * (1.0 / TAU)
        col = lax.broadcasted_iota(jnp.int32, (BF, 32), 1)
        logits = jnp.where(col < K, logits, -1e30)
        m = jnp.max(logits, axis=-1, keepdims=True)
        e = jnp.exp(logits - m)
        coeff = e / jnp.sum(e, axis=-1, keepdims=True)
        hf0 = jnp.maximum(jnp.dot(geo16, w0_ref[...], precision=_HIGH), 0.0)
        scale = jnp.dot(coeff, wm_ref[...], precision=_HIGH)
        t0_ref[...] = hf0 * scale
        cf_ref[...] = coeff

    grid = (F_PAD // BF,)
    return pl.pallas_call(
        body,
        grid=grid,
        in_specs=[
            pl.BlockSpec((BF, 8), lambda i: (i, 0)),
            pl.BlockSpec((BF, 8), lambda i: (i, 0)),
            pl.BlockSpec((BF, 8), lambda i: (i, 0)),
            pl.BlockSpec((8, 32), lambda i: (0, 0)),
            pl.BlockSpec((16, 32), lambda i: (0, 0)),
            pl.BlockSpec((32, 32), lambda i: (0, 0)),
        ],
        out_specs=[
            pl.BlockSpec((BF, 32), lambda i: (i, 0)),
            pl.BlockSpec((BF, 32), lambda i: (i, 0)),
        ],
        out_shape=[
            jax.ShapeDtypeStruct((F_PAD, 32), jnp.float32),
            jax.ShapeDtypeStruct((F_PAD, 32), jnp.float32),
        ],
    )(va, vb, vc, centers_p, w0_p, wm0_p)


def _tc_layer(ga, gb, gc, coeff, wv_s, wm_p):
    BF = 256
    cin = ga.shape[1]
    cout = wv_s.shape[1]

    def body(ga_ref, gb_ref, gc_ref, cf_ref, wv_ref, wm_ref, t_ref):
        g = ga_ref[...] + gb_ref[...] + gc_ref[...]
        hf = jnp.maximum(jnp.dot(g, wv_ref[...]), 0.0)
        scale = jnp.dot(cf_ref[...], wm_ref[...])
        t_ref[...] = hf * scale

    return pl.pallas_call(
        body,
        grid=(F_PAD // BF,),
        in_specs=[
            pl.BlockSpec((BF, cin), lambda i: (i, 0)),
            pl.BlockSpec((BF, cin), lambda i: (i, 0)),
            pl.BlockSpec((BF, cin), lambda i: (i, 0)),
            pl.BlockSpec((BF, 32), lambda i: (i, 0)),
            pl.BlockSpec((cin, cout), lambda i: (0, 0)),
            pl.BlockSpec((32, cout), lambda i: (0, 0)),
        ],
        out_specs=pl.BlockSpec((BF, cout), lambda i: (i, 0)),
        out_shape=jax.ShapeDtypeStruct((F_PAD, cout), jnp.float32),
    )(ga, gb, gc, coeff, wv_s, wm_p)


def _tc_pool(feats3):
    seg = N // B

    def body(f_ref, o_ref):
        o_ref[0, 0, :] = jnp.sum(f_ref[0], axis=0) * (1.0 / seg)

    return pl.pallas_call(
        body,
        grid=(B,),
        in_specs=[pl.BlockSpec((1, seg, 256), lambda i: (i, 0, 0))],
        out_specs=pl.BlockSpec((1, 1, 256), lambda i: (i, 0, 0)),
        out_shape=jax.ShapeDtypeStruct((B, 1, 256), jnp.float32),
    )(feats3)


def _tc_fc(pooled, w1, b1, w2, b2):
    def body(p_ref, w1_ref, b1_ref, w2_ref, b2_ref, o_ref):
        h = jnp.maximum(
            jnp.dot(p_ref[...], w1_ref[...], precision=_HIGH) + b1_ref[...], 0.0)
        o_ref[...] = jnp.dot(h, w2_ref[...], precision=_HIGH) + b2_ref[...]

    return pl.pallas_call(
        body,
        out_shape=jax.ShapeDtypeStruct((B, 20), jnp.float32),
    )(pooled, w1, b1, w2, b2)


def kernel(vertex_in, face_in, nv_in, mf_in, centers, W0, Wm0, Wv0, Wv1, Wv2,
           Wv3, Wv4, Wmb0, Wmb1, Wmb2, Wmb3, Wmb4, W_fc1, b_fc1, W_fc2, b_fc2):
    face32 = face_in.astype(jnp.int32)
    pad_idx = jnp.full((F_PAD - F, 3), JUNK, jnp.int32)
    fidx = jnp.concatenate([face32, pad_idx], axis=0).T.reshape(3, NBLK, FB)
    vert_p = jnp.pad(vertex_in, ((0, N_PAD - N), (0, 5)))

    centers_p = jnp.zeros((8, 32), jnp.float32).at[0:3, 0:K].set(centers.T)
    w0_p = jnp.pad(W0, ((0, 7), (0, 0)))
    wm0_p = jnp.pad(Wm0, ((0, 32 - K), (0, 0)))
    wvs = [w / 3.0 for w in (Wv0, Wv1, Wv2, Wv3, Wv4)]
    wms = [jnp.pad(w, ((0, 32 - K), (0, 0))) for w in (Wmb0, Wmb1, Wmb2, Wmb3, Wmb4)]

    g3v, cnt3 = _sc_gather3(vert_p, fidx, with_count=True)
    inv = _tc_inv(cnt3[:, :, 0].reshape(2, 392, 128)).reshape(N_PAD)
    t0, coeff = _tc_geom(g3v[0], g3v[1], g3v[2], centers_p, w0_p, wm0_p)

    feats = _sc_scatter3(t0, fidx, inv, CH[0])
    for l in range(5):
        g3 = _sc_gather3(feats, fidx)
        t = _tc_layer(g3[0], g3[1], g3[2], coeff, wvs[l], wms[l])
        feats = _sc_scatter3(t, fidx, inv, CH[l + 1])

    pooled = _tc_pool(feats[:N].reshape(B, N // B, 256)).reshape(B, 256)
    logits = _tc_fc(pooled, W_fc1, b_fc1.reshape(1, 128), W_fc2,
                    b_fc2.reshape(1, 20))
    return logits
